# Initial kernel scaffold; baseline (speedup 1.0000x reference)
import jax
import jax.numpy as jnp
from jax.experimental import pallas as pl


def kernel(pos, edge_index, W_emb, b_emb, W_theta, b_theta, W_phi, b_phi, W_e1, b_e1, W_e2, b_e2, W_ih, b_ih, W_hh, b_hh, W_out, b_out):
    raise NotImplementedError("write your pallas kernel here")


# speedup vs baseline: 2.0606x; 2.0606x over previous
import functools

import jax
import jax.numpy as jnp
from jax import lax
from jax.experimental import pallas as pl
from jax.experimental.pallas import tpu as pltpu
from jax.experimental.pallas import tpu_sc as plsc

N_NODES = 10000
NP = 10240
E = 160000
NC, NS = 2, 16
NW = NC * NS
E_PAD = 163840
PER_W = E_PAD // NW
CHUNK = 128
NCHUNK = PER_W // CHUNK
H = 32
DP = 128
ZROWS = NP // NS


@functools.cache
def _sc_kernels():
    mesh = plsc.VectorSubcoreMesh(core_axis_name="c", subcore_axis_name="s",
                                  num_cores=NC)

    @functools.partial(
        pl.kernel, mesh=mesh,
        out_type=[jax.ShapeDtypeStruct((E_PAD, DP), jnp.float32),
                  jax.ShapeDtypeStruct((E_PAD, DP), jnp.float32)],
        scratch_types=[pltpu.VMEM((CHUNK,), jnp.int32),
                       pltpu.VMEM((CHUNK, DP), jnp.float32),
                       pltpu.SemaphoreType.DMA],
    )
    def gather2(h_hbm, src_hbm, dst_hbm, xs_hbm, xd_hbm, idx_v, rows_v, sem):
        wid = lax.axis_index("s") * NC + lax.axis_index("c")
        base = wid * PER_W

        def body(j, carry):
            off = base + j * CHUNK
            pltpu.sync_copy(src_hbm.at[pl.ds(off, CHUNK)], idx_v)
            pltpu.async_copy(h_hbm.at[idx_v], rows_v, sem).wait()
            pltpu.sync_copy(rows_v, xs_hbm.at[pl.ds(off, CHUNK)])
            pltpu.sync_copy(dst_hbm.at[pl.ds(off, CHUNK)], idx_v)
            pltpu.async_copy(h_hbm.at[idx_v], rows_v, sem).wait()
            pltpu.sync_copy(rows_v, xd_hbm.at[pl.ds(off, CHUNK)])
            return carry

        lax.fori_loop(0, NCHUNK, body, 0)

    @functools.partial(
        pl.kernel, mesh=mesh,
        out_type=jax.ShapeDtypeStruct((E_PAD, DP), jnp.float32),
        scratch_types=[pltpu.VMEM((CHUNK,), jnp.int32),
                       pltpu.VMEM((CHUNK, DP), jnp.float32),
                       pltpu.SemaphoreType.DMA],
    )
    def gather1(h_hbm, src_hbm, xs_hbm, idx_v, rows_v, sem):
        wid = lax.axis_index("s") * NC + lax.axis_index("c")
        base = wid * PER_W

        def body(j, carry):
            off = base + j * CHUNK
            pltpu.sync_copy(src_hbm.at[pl.ds(off, CHUNK)], idx_v)
            pltpu.async_copy(h_hbm.at[idx_v], rows_v, sem).wait()
            pltpu.sync_copy(rows_v, xs_hbm.at[pl.ds(off, CHUNK)])
            return carry

        lax.fori_loop(0, NCHUNK, body, 0)

    @functools.partial(
        pl.kernel, mesh=mesh,
        out_type=jax.ShapeDtypeStruct((NC, NP, DP), jnp.float32),
        scratch_types=[pltpu.VMEM((CHUNK,), jnp.int32),
                       pltpu.VMEM((CHUNK, DP), jnp.float32),
                       pltpu.VMEM_SHARED((NP, DP), jnp.float32),
                       pltpu.SemaphoreType.DMA],
    )
    def scatter(m_hbm, dst_hbm, z_hbm, out_hbm, idx_v, rows_v, acc, sem):
        c = lax.axis_index("c")
        s = lax.axis_index("s")
        wid = s * NC + c
        base = wid * PER_W
        pltpu.sync_copy(z_hbm, acc.at[pl.ds(s * ZROWS, ZROWS)])
        plsc.subcore_barrier()

        def body(j, carry):
            off = base + j * CHUNK
            pltpu.sync_copy(dst_hbm.at[pl.ds(off, CHUNK)], idx_v)
            pltpu.sync_copy(m_hbm.at[pl.ds(off, CHUNK)], rows_v)
            pltpu.sync_copy(rows_v, acc.at[idx_v], add=True)
            return carry

        lax.fori_loop(0, NCHUNK, body, 0)
        plsc.subcore_barrier()
        pltpu.sync_copy(acc.at[pl.ds(s * ZROWS, ZROWS)],
                        out_hbm.at[c, pl.ds(s * ZROWS, ZROWS)])

    return gather2, gather1, scatter


def _embed_body(pos_ref, w_ref, b_ref, o_ref):
    h = (jnp.dot(pos_ref[...], w_ref[...],
                 preferred_element_type=jnp.float32) + b_ref[...])
    o_ref[...] = jnp.pad(h, ((0, 0), (0, DP - H)))


def _embed(pos_p, W_emb, b_emb):
    bn = 2048
    return pl.pallas_call(
        _embed_body,
        grid=(NP // bn,),
        in_specs=[pl.BlockSpec((bn, 128), lambda i: (i, 0)),
                  pl.BlockSpec((128, H), lambda i: (0, 0)),
                  pl.BlockSpec((1, H), lambda i: (0, 0))],
        out_specs=pl.BlockSpec((bn, DP), lambda i: (i, 0)),
        out_shape=jax.ShapeDtypeStruct((NP, DP), jnp.float32),
    )(pos_p, W_emb, b_emb.reshape(1, H))


def _edgefeat_body(xs_ref, xd_ref, wt_ref, bt_ref, wp_ref, bp_ref,
                   we1_ref, be1_ref, aT_ref):
    xs = xs_ref[:, :H]
    xd = xd_ref[:, :H]
    he = jnp.dot(xd - xs, wt_ref[...], preferred_element_type=jnp.float32)
    he = he + jnp.dot(xs, wp_ref[...], preferred_element_type=jnp.float32)
    he = jnp.maximum(he + bt_ref[...] + bp_ref[...], 0.0)
    a = jnp.maximum(jnp.dot(he, we1_ref[...],
                            preferred_element_type=jnp.float32) + be1_ref[...],
                    0.0)
    aT_ref[...] = a.T


def _edgefeat(xs, xd, W_theta, b_theta, W_phi, b_phi, W_e1, b_e1):
    be = 2048
    return pl.pallas_call(
        _edgefeat_body,
        grid=(E_PAD // be,),
        in_specs=[pl.BlockSpec((be, DP), lambda i: (i, 0)),
                  pl.BlockSpec((be, DP), lambda i: (i, 0)),
                  pl.BlockSpec((H, H), lambda i: (0, 0)),
                  pl.BlockSpec((1, H), lambda i: (0, 0)),
                  pl.BlockSpec((H, H), lambda i: (0, 0)),
                  pl.BlockSpec((1, H), lambda i: (0, 0)),
                  pl.BlockSpec((H, 64), lambda i: (0, 0)),
                  pl.BlockSpec((1, 64), lambda i: (0, 0))],
        out_specs=pl.BlockSpec((64, be), lambda i: (0, i)),
        out_shape=jax.ShapeDtypeStruct((64, E_PAD), jnp.float32),
    )(xs, xd, W_theta, b_theta.reshape(1, H), W_phi, b_phi.reshape(1, H),
      W_e1, b_e1.reshape(1, 64))


def _msg_body(aT_ref, xs_ref, w2T_ref, b2_ref, m_ref):
    be = xs_ref.shape[0]
    weT = jnp.dot(w2T_ref[...], aT_ref[...],
                  preferred_element_type=jnp.float32) + b2_ref[...]
    weT = weT.astype(jnp.bfloat16).astype(jnp.float32)
    xT = xs_ref[:, :H].T
    xT = xT.astype(jnp.bfloat16).astype(jnp.float32)
    v = weT.reshape(H, H, be) * xT[None, :, :]
    mT = v.sum(axis=1)
    m_ref[...] = jnp.pad(mT.T, ((0, 0), (0, DP - H)))


def _msg(aT, xs, W_e2T, b_e2col):
    be = 1024
    return pl.pallas_call(
        _msg_body,
        grid=(E_PAD // be,),
        in_specs=[pl.BlockSpec((64, be), lambda i: (0, i)),
                  pl.BlockSpec((be, DP), lambda i: (i, 0)),
                  pl.BlockSpec((H * H, 64), lambda i: (0, 0)),
                  pl.BlockSpec((H * H, 1), lambda i: (0, 0))],
        out_specs=pl.BlockSpec((be, DP), lambda i: (i, 0)),
        out_shape=jax.ShapeDtypeStruct((E_PAD, DP), jnp.float32),
    )(aT, xs, W_e2T, b_e2col)


def _gru_body(p0_ref, p1_ref, h_ref, wih_ref, bih_ref, whh_ref, bhh_ref,
              ho_ref):
    m = p0_ref[:, :H] + p1_ref[:, :H]
    gi = jnp.dot(m, wih_ref[...], preferred_element_type=jnp.float32) \
        + bih_ref[...]
    h = h_ref[:, :H]
    gh = jnp.dot(h, whh_ref[...], preferred_element_type=jnp.float32) \
        + bhh_ref[...]
    r = jax.nn.sigmoid(gi[:, :H] + gh[:, :H])
    z = jax.nn.sigmoid(gi[:, H:2 * H] + gh[:, H:2 * H])
    n = jnp.tanh(gi[:, 2 * H:] + r * gh[:, 2 * H:])
    hn = (1.0 - z) * n + z * h
    ho_ref[...] = jnp.pad(hn, ((0, 0), (0, DP - H)))


def _gru(parts, h, W_ih, b_ih, W_hh, b_hh):
    bn = 2048
    return pl.pallas_call(
        _gru_body,
        grid=(NP // bn,),
        in_specs=[pl.BlockSpec((bn, DP), lambda i: (i, 0)),
                  pl.BlockSpec((bn, DP), lambda i: (i, 0)),
                  pl.BlockSpec((bn, DP), lambda i: (i, 0)),
                  pl.BlockSpec((H, 3 * H), lambda i: (0, 0)),
                  pl.BlockSpec((1, 3 * H), lambda i: (0, 0)),
                  pl.BlockSpec((H, 3 * H), lambda i: (0, 0)),
                  pl.BlockSpec((1, 3 * H), lambda i: (0, 0))],
        out_specs=pl.BlockSpec((bn, DP), lambda i: (i, 0)),
        out_shape=jax.ShapeDtypeStruct((NP, DP), jnp.float32),
    )(parts[0], parts[1], h, W_ih, b_ih.reshape(1, 3 * H),
      W_hh, b_hh.reshape(1, 3 * H))


def _proj_body(h_ref, w_ref, b_ref, o_ref):
    o_ref[...] = (jnp.dot(h_ref[:, :H], w_ref[...],
                          preferred_element_type=jnp.float32) + b_ref[...])


def _proj(h, W_out, b_out):
    bn = 2048
    return pl.pallas_call(
        _proj_body,
        grid=(NP // bn,),
        in_specs=[pl.BlockSpec((bn, DP), lambda i: (i, 0)),
                  pl.BlockSpec((H, H), lambda i: (0, 0)),
                  pl.BlockSpec((1, H), lambda i: (0, 0))],
        out_specs=pl.BlockSpec((bn, H), lambda i: (i, 0)),
        out_shape=jax.ShapeDtypeStruct((NP, H), jnp.float32),
    )(h, W_out, b_out.reshape(1, H))


def kernel(pos, edge_index, W_emb, b_emb, W_theta, b_theta, W_phi, b_phi,
           W_e1, b_e1, W_e2, b_e2, W_ih, b_ih, W_hh, b_hh, W_out, b_out):
    src = edge_index[0]
    dst = edge_index[1]
    pad = E_PAD - E
    src_p = jnp.concatenate([src, jnp.zeros((pad,), jnp.int32)])
    dst_p = jnp.concatenate([dst, jnp.full((pad,), N_NODES, jnp.int32)])
    pos_p = jnp.concatenate(
        [pos, jnp.zeros((NP - N_NODES, pos.shape[1]), jnp.float32)])
    W_e2T = W_e2.T
    b_e2col = b_e2.reshape(H * H, 1)
    zstripe = jnp.zeros((ZROWS, DP), jnp.float32)

    sc_gather2, sc_gather1, sc_scatter = _sc_kernels()

    h = _embed(pos_p, W_emb, b_emb)
    xs, xd = sc_gather2(h, src_p, dst_p)
    aT = _edgefeat(xs, xd, W_theta, b_theta, W_phi, b_phi, W_e1, b_e1)

    for step in range(3):
        if step > 0:
            xs = sc_gather1(h, src_p)
        m_e = _msg(aT, xs, W_e2T, b_e2col)
        parts = sc_scatter(m_e, dst_p, zstripe)
        h = _gru(parts, h, W_ih, b_ih, W_hh, b_hh)

    out = _proj(h, W_out, b_out)
    return out[:N_NODES]


# speedup vs baseline: 2.7428x; 1.3311x over previous
import functools

import jax
import jax.numpy as jnp
from jax import lax
from jax.experimental import pallas as pl
from jax.experimental.pallas import tpu as pltpu
from jax.experimental.pallas import tpu_sc as plsc

N_NODES = 10000
NP = 10240
E = 160000
NC, NS = 2, 16
NW = NC * NS
E_PAD = 163840
PER_W = E_PAD // NW
CHUNK = 128
NCHUNK = PER_W // CHUNK
H = 32
DP = 128
ZROWS = NP // NS


@functools.cache
def _sc_kernels():
    mesh = plsc.VectorSubcoreMesh(core_axis_name="c", subcore_axis_name="s",
                                  num_cores=NC)

    @functools.partial(
        pl.kernel, mesh=mesh,
        out_type=jax.ShapeDtypeStruct((E_PAD, DP), jnp.float32),
        scratch_types=[pltpu.VMEM((NCHUNK, CHUNK), jnp.int32),
                       pltpu.VMEM((CHUNK, DP), jnp.float32),
                       pltpu.VMEM((CHUNK, DP), jnp.float32),
                       pltpu.SemaphoreType.DMA,
                       pltpu.SemaphoreType.DMA],
    )
    def gather1(h_hbm, src_hbm, xs_hbm, idx_v, rows0, rows1, sem0, sem1):
        wid = lax.axis_index("s") * NC + lax.axis_index("c")
        base = wid * PER_W
        pltpu.sync_copy(src_hbm.at[pl.ds(wid * NCHUNK, NCHUNK)], idx_v)
        bufs = ((rows0, sem0), (rows1, sem1))

        def fire(j, b):
            rows, sem = bufs[b]
            pltpu.async_copy(h_hbm.at[idx_v.at[j]], rows, sem)

        def drain_wb(j, b):
            rows, sem = bufs[b]
            pltpu.make_async_copy(h_hbm.at[idx_v.at[j]], rows, sem).wait()
            pltpu.sync_copy(rows, xs_hbm.at[pl.ds(base + j * CHUNK, CHUNK)])

        fire(0, 0)
        fire(1, 1)

        def body(g, carry):
            j = 2 * g
            drain_wb(j, 0)
            fire(j + 2, 0)
            drain_wb(j + 1, 1)
            fire(j + 3, 1)
            return carry

        lax.fori_loop(0, NCHUNK // 2 - 1, body, 0)
        drain_wb(NCHUNK - 2, 0)
        drain_wb(NCHUNK - 1, 1)

    @functools.partial(
        pl.kernel, mesh=mesh,
        out_type=[jax.ShapeDtypeStruct((E_PAD, DP), jnp.float32),
                  jax.ShapeDtypeStruct((E_PAD, DP), jnp.float32)],
        scratch_types=[pltpu.VMEM((NCHUNK, CHUNK), jnp.int32),
                       pltpu.VMEM((NCHUNK, CHUNK), jnp.int32),
                       pltpu.VMEM((CHUNK, DP), jnp.float32),
                       pltpu.VMEM((CHUNK, DP), jnp.float32),
                       pltpu.VMEM((CHUNK, DP), jnp.float32),
                       pltpu.VMEM((CHUNK, DP), jnp.float32),
                       pltpu.SemaphoreType.DMA,
                       pltpu.SemaphoreType.DMA,
                       pltpu.SemaphoreType.DMA,
                       pltpu.SemaphoreType.DMA],
    )
    def gather2(h_hbm, src_hbm, dst_hbm, xs_hbm, xd_hbm, idxs_v, idxd_v,
                rs0, rs1, rd0, rd1, sems0, sems1, semd0, semd1):
        wid = lax.axis_index("s") * NC + lax.axis_index("c")
        base = wid * PER_W
        pltpu.sync_copy(src_hbm.at[pl.ds(wid * NCHUNK, NCHUNK)], idxs_v)
        pltpu.sync_copy(dst_hbm.at[pl.ds(wid * NCHUNK, NCHUNK)], idxd_v)
        sbufs = ((rs0, sems0), (rs1, sems1))
        dbufs = ((rd0, semd0), (rd1, semd1))

        def fire(bufs, idx, j, b):
            rows, sem = bufs[b]
            pltpu.async_copy(h_hbm.at[idx.at[j]], rows, sem)

        def drain_wb(bufs, idx, out, j, b):
            rows, sem = bufs[b]
            pltpu.make_async_copy(h_hbm.at[idx.at[j]], rows, sem).wait()
            pltpu.sync_copy(rows, out.at[pl.ds(base + j * CHUNK, CHUNK)])

        for b in (0, 1):
            fire(sbufs, idxs_v, b, b)
            fire(dbufs, idxd_v, b, b)

        def body(g, carry):
            j = 2 * g
            drain_wb(sbufs, idxs_v, xs_hbm, j, 0)
            fire(sbufs, idxs_v, j + 2, 0)
            drain_wb(dbufs, idxd_v, xd_hbm, j, 0)
            fire(dbufs, idxd_v, j + 2, 0)
            drain_wb(sbufs, idxs_v, xs_hbm, j + 1, 1)
            fire(sbufs, idxs_v, j + 3, 1)
            drain_wb(dbufs, idxd_v, xd_hbm, j + 1, 1)
            fire(dbufs, idxd_v, j + 3, 1)
            return carry

        lax.fori_loop(0, NCHUNK // 2 - 1, body, 0)
        for j, b in ((NCHUNK - 2, 0), (NCHUNK - 1, 1)):
            drain_wb(sbufs, idxs_v, xs_hbm, j, b)
            drain_wb(dbufs, idxd_v, xd_hbm, j, b)

    @functools.partial(
        pl.kernel, mesh=mesh,
        out_type=jax.ShapeDtypeStruct((NC, NP, DP), jnp.float32),
        scratch_types=[pltpu.VMEM((NCHUNK, CHUNK), jnp.int32),
                       pltpu.VMEM((CHUNK, DP), jnp.float32),
                       pltpu.VMEM((CHUNK, DP), jnp.float32),
                       pltpu.VMEM_SHARED((NP, DP), jnp.float32),
                       pltpu.SemaphoreType.DMA,
                       pltpu.SemaphoreType.DMA],
    )
    def scatter(m_hbm, dst_hbm, z_hbm, out_hbm, idx_v, rows0, rows1, acc,
                sem0, sem1):
        c = lax.axis_index("c")
        s = lax.axis_index("s")
        wid = s * NC + c
        base = wid * PER_W
        pltpu.sync_copy(dst_hbm.at[pl.ds(wid * NCHUNK, NCHUNK)], idx_v)
        pltpu.sync_copy(z_hbm, acc.at[pl.ds(s * ZROWS, ZROWS)])
        plsc.subcore_barrier()
        bufs = ((rows0, sem0), (rows1, sem1))

        def fire(j, b):
            rows, sem = bufs[b]
            pltpu.async_copy(m_hbm.at[pl.ds(base + j * CHUNK, CHUNK)], rows,
                             sem)

        def drain_add(j, b):
            rows, sem = bufs[b]
            pltpu.make_async_copy(
                m_hbm.at[pl.ds(base + j * CHUNK, CHUNK)], rows, sem).wait()
            pltpu.sync_copy(rows, acc.at[idx_v.at[j]], add=True)

        fire(0, 0)
        fire(1, 1)

        def body(g, carry):
            j = 2 * g
            drain_add(j, 0)
            fire(j + 2, 0)
            drain_add(j + 1, 1)
            fire(j + 3, 1)
            return carry

        lax.fori_loop(0, NCHUNK // 2 - 1, body, 0)
        drain_add(NCHUNK - 2, 0)
        drain_add(NCHUNK - 1, 1)
        plsc.subcore_barrier()
        pltpu.sync_copy(acc.at[pl.ds(s * ZROWS, ZROWS)],
                        out_hbm.at[c, pl.ds(s * ZROWS, ZROWS)])

    return gather2, gather1, scatter


def _embed_body(pos_ref, w_ref, b_ref, o_ref):
    h = (jnp.dot(pos_ref[...], w_ref[...],
                 preferred_element_type=jnp.float32) + b_ref[...])
    o_ref[...] = jnp.pad(h, ((0, 0), (0, DP - H)))


def _embed(pos_p, W_emb, b_emb):
    bn = 2048
    return pl.pallas_call(
        _embed_body,
        grid=(NP // bn,),
        in_specs=[pl.BlockSpec((bn, 128), lambda i: (i, 0)),
                  pl.BlockSpec((128, H), lambda i: (0, 0)),
                  pl.BlockSpec((1, H), lambda i: (0, 0))],
        out_specs=pl.BlockSpec((bn, DP), lambda i: (i, 0)),
        out_shape=jax.ShapeDtypeStruct((NP, DP), jnp.float32),
    )(pos_p, W_emb, b_emb.reshape(1, H))


def _edgefeat_body(xs_ref, xd_ref, wt_ref, bt_ref, wp_ref, bp_ref,
                   we1_ref, be1_ref, aT_ref):
    xs = xs_ref[:, :H]
    xd = xd_ref[:, :H]
    he = jnp.dot(xd - xs, wt_ref[...], preferred_element_type=jnp.float32)
    he = he + jnp.dot(xs, wp_ref[...], preferred_element_type=jnp.float32)
    he = jnp.maximum(he + bt_ref[...] + bp_ref[...], 0.0)
    a = jnp.maximum(jnp.dot(he, we1_ref[...],
                            preferred_element_type=jnp.float32) + be1_ref[...],
                    0.0)
    aT_ref[...] = a.T


def _edgefeat(xs, xd, W_theta, b_theta, W_phi, b_phi, W_e1, b_e1):
    be = 2048
    return pl.pallas_call(
        _edgefeat_body,
        grid=(E_PAD // be,),
        in_specs=[pl.BlockSpec((be, DP), lambda i: (i, 0)),
                  pl.BlockSpec((be, DP), lambda i: (i, 0)),
                  pl.BlockSpec((H, H), lambda i: (0, 0)),
                  pl.BlockSpec((1, H), lambda i: (0, 0)),
                  pl.BlockSpec((H, H), lambda i: (0, 0)),
                  pl.BlockSpec((1, H), lambda i: (0, 0)),
                  pl.BlockSpec((H, 64), lambda i: (0, 0)),
                  pl.BlockSpec((1, 64), lambda i: (0, 0))],
        out_specs=pl.BlockSpec((64, be), lambda i: (0, i)),
        out_shape=jax.ShapeDtypeStruct((64, E_PAD), jnp.float32),
    )(xs, xd, W_theta, b_theta.reshape(1, H), W_phi, b_phi.reshape(1, H),
      W_e1, b_e1.reshape(1, 64))


def _msg_body(aT_ref, xs_ref, w2T_ref, b2_ref, m_ref):
    be = xs_ref.shape[0]
    weT = jnp.dot(w2T_ref[...], aT_ref[...],
                  preferred_element_type=jnp.float32) + b2_ref[...]
    weT = weT.astype(jnp.bfloat16).astype(jnp.float32)
    xT = xs_ref[:, :H].T
    xT = xT.astype(jnp.bfloat16).astype(jnp.float32)
    v = weT.reshape(H, H, be) * xT[None, :, :]
    mT = v.sum(axis=1)
    m_ref[...] = jnp.pad(mT.T, ((0, 0), (0, DP - H)))


def _msg(aT, xs, W_e2T, b_e2col):
    be = 1024
    return pl.pallas_call(
        _msg_body,
        grid=(E_PAD // be,),
        in_specs=[pl.BlockSpec((64, be), lambda i: (0, i)),
                  pl.BlockSpec((be, DP), lambda i: (i, 0)),
                  pl.BlockSpec((H * H, 64), lambda i: (0, 0)),
                  pl.BlockSpec((H * H, 1), lambda i: (0, 0))],
        out_specs=pl.BlockSpec((be, DP), lambda i: (i, 0)),
        out_shape=jax.ShapeDtypeStruct((E_PAD, DP), jnp.float32),
    )(aT, xs, W_e2T, b_e2col)


def _gru_body(p0_ref, p1_ref, h_ref, wih_ref, bih_ref, whh_ref, bhh_ref,
              ho_ref):
    m = p0_ref[:, :H] + p1_ref[:, :H]
    gi = jnp.dot(m, wih_ref[...], preferred_element_type=jnp.float32) \
        + bih_ref[...]
    h = h_ref[:, :H]
    gh = jnp.dot(h, whh_ref[...], preferred_element_type=jnp.float32) \
        + bhh_ref[...]
    r = jax.nn.sigmoid(gi[:, :H] + gh[:, :H])
    z = jax.nn.sigmoid(gi[:, H:2 * H] + gh[:, H:2 * H])
    n = jnp.tanh(gi[:, 2 * H:] + r * gh[:, 2 * H:])
    hn = (1.0 - z) * n + z * h
    ho_ref[...] = jnp.pad(hn, ((0, 0), (0, DP - H)))


def _gru(parts, h, W_ih, b_ih, W_hh, b_hh):
    bn = 2048
    return pl.pallas_call(
        _gru_body,
        grid=(NP // bn,),
        in_specs=[pl.BlockSpec((bn, DP), lambda i: (i, 0)),
                  pl.BlockSpec((bn, DP), lambda i: (i, 0)),
                  pl.BlockSpec((bn, DP), lambda i: (i, 0)),
                  pl.BlockSpec((H, 3 * H), lambda i: (0, 0)),
                  pl.BlockSpec((1, 3 * H), lambda i: (0, 0)),
                  pl.BlockSpec((H, 3 * H), lambda i: (0, 0)),
                  pl.BlockSpec((1, 3 * H), lambda i: (0, 0))],
        out_specs=pl.BlockSpec((bn, DP), lambda i: (i, 0)),
        out_shape=jax.ShapeDtypeStruct((NP, DP), jnp.float32),
    )(parts[0], parts[1], h, W_ih, b_ih.reshape(1, 3 * H),
      W_hh, b_hh.reshape(1, 3 * H))


def _proj_body(h_ref, w_ref, b_ref, o_ref):
    o_ref[...] = (jnp.dot(h_ref[:, :H], w_ref[...],
                          preferred_element_type=jnp.float32) + b_ref[...])


def _proj(h, W_out, b_out):
    bn = 2048
    return pl.pallas_call(
        _proj_body,
        grid=(NP // bn,),
        in_specs=[pl.BlockSpec((bn, DP), lambda i: (i, 0)),
                  pl.BlockSpec((H, H), lambda i: (0, 0)),
                  pl.BlockSpec((1, H), lambda i: (0, 0))],
        out_specs=pl.BlockSpec((bn, H), lambda i: (i, 0)),
        out_shape=jax.ShapeDtypeStruct((NP, H), jnp.float32),
    )(h, W_out, b_out.reshape(1, H))


def kernel(pos, edge_index, W_emb, b_emb, W_theta, b_theta, W_phi, b_phi,
           W_e1, b_e1, W_e2, b_e2, W_ih, b_ih, W_hh, b_hh, W_out, b_out):
    src = edge_index[0]
    dst = edge_index[1]
    pad = E_PAD - E
    src_p = jnp.concatenate([src, jnp.zeros((pad,), jnp.int32)])
    dst_p = jnp.concatenate([dst, jnp.full((pad,), N_NODES, jnp.int32)])
    src_2d = src_p.reshape(E_PAD // CHUNK, CHUNK)
    dst_2d = dst_p.reshape(E_PAD // CHUNK, CHUNK)
    pos_p = jnp.concatenate(
        [pos, jnp.zeros((NP - N_NODES, pos.shape[1]), jnp.float32)])
    W_e2T = W_e2.T
    b_e2col = b_e2.reshape(H * H, 1)
    zstripe = jnp.zeros((ZROWS, DP), jnp.float32)

    sc_gather2, sc_gather1, sc_scatter = _sc_kernels()

    h = _embed(pos_p, W_emb, b_emb)
    xs, xd = sc_gather2(h, src_2d, dst_2d)
    aT = _edgefeat(xs, xd, W_theta, b_theta, W_phi, b_phi, W_e1, b_e1)

    for step in range(3):
        if step > 0:
            xs = sc_gather1(h, src_2d)
        m_e = _msg(aT, xs, W_e2T, b_e2col)
        parts = sc_scatter(m_e, dst_2d, zstripe)
        h = _gru(parts, h, W_ih, b_ih, W_hh, b_hh)

    out = _proj(h, W_out, b_out)
    return out[:N_NODES]
